# baseline (speedup 1.0000x reference)
import jax
import jax.numpy as jnp
from jax.experimental import pallas as pl

N = 10000
D_IN = 128
D_OUT = 128
K = 136
BM = 200


def _gcn_kernel(adj_ref, x1_ref, wb_ref, out_ref):
    a = adj_ref[...].astype(jnp.bfloat16)
    t = jnp.dot(a, x1_ref[...], preferred_element_type=jnp.float32)
    out_ref[...] = jnp.dot(
        t, wb_ref[...], preferred_element_type=jnp.float32
    )


@jax.jit
def kernel(x, adj, W, b):
    x1 = jnp.concatenate(
        [
            x.astype(jnp.bfloat16),
            jnp.ones((N, 1), jnp.bfloat16),
            jnp.zeros((N, K - D_IN - 1), jnp.bfloat16),
        ],
        axis=1,
    )
    wb = jnp.concatenate(
        [W.T, b.reshape(1, D_OUT), jnp.zeros((K - D_IN - 1, D_OUT), jnp.float32)],
        axis=0,
    )

    out = pl.pallas_call(
        _gcn_kernel,
        grid=(N // BM,),
        in_specs=[
            pl.BlockSpec((BM, N), lambda i: (i, 0)),
            pl.BlockSpec((N, K), lambda i: (0, 0)),
            pl.BlockSpec((K, D_OUT), lambda i: (0, 0)),
        ],
        out_specs=pl.BlockSpec((BM, D_OUT), lambda i: (i, 0)),
        out_shape=jax.ShapeDtypeStruct((N, D_OUT), jnp.float32),
    )(adj, x1, wb)
    return out
